# baseline (speedup 1.0000x reference)
import functools

import jax
import jax.numpy as jnp
from jax import lax
from jax.experimental import pallas as pl
from jax.experimental.pallas import tpu as pltpu
from jax.experimental.pallas import tpu_sc as plsc

B = 16384
N_FIELDS = 26
VOCAB = 100000
EMB = 64
NUM_NUM = 13

K_EMB = N_FIELDS * EMB
K_HALF = K_EMB // 2
NW = 32
P_PER_W = K_HALF // NW
BCH = 8192
N_BCH = B // BCH
UNROLL = 8


@functools.cache
def _sc_gather_fn(lo):
    mesh = plsc.VectorSubcoreMesh(core_axis_name="c", subcore_axis_name="s")

    @functools.partial(
        pl.kernel,
        out_type=jax.ShapeDtypeStruct((K_HALF, B), jnp.float32),
        mesh=mesh,
        compiler_params=pltpu.CompilerParams(needs_layout_passes=False),
        scratch_types=[
            pltpu.VMEM((VOCAB,), jnp.float32),
            pltpu.VMEM((B,), jnp.int32),
            pltpu.VMEM((BCH,), jnp.float32),
        ],
    )
    def _sc_gather(xct_hbm, table_hbm, out_hbm, plane_v, idx_v, val_v):
        wid = lax.axis_index("s") * 2 + lax.axis_index("c")
        p_base = wid * P_PER_W

        def plane(p, prev_f):
            fe = lo + p_base + p
            f = fe // EMB

            @pl.when(f != prev_f)
            def _():
                pltpu.sync_copy(xct_hbm.at[f], idx_v)

            pltpu.sync_copy(table_hbm.at[fe], plane_v)

            def bchunk(k, carry2):
                koff = pl.multiple_of(k * BCH, BCH)

                def group(g, carry3):
                    base = pl.multiple_of(g * (16 * UNROLL), 16 * UNROLL)
                    for j in range(UNROLL):
                        o = base + j * 16
                        vidx = idx_v[pl.ds(koff + o, 16)]
                        v = plsc.load_gather(plane_v, [vidx])
                        val_v[pl.ds(o, 16)] = v
                    return carry3

                lax.fori_loop(0, BCH // (16 * UNROLL), group, 0)
                pltpu.sync_copy(val_v,
                                out_hbm.at[fe - lo, pl.ds(koff, BCH)])
                return carry2

            lax.fori_loop(0, N_BCH, bchunk, 0)
            return f

        lax.fori_loop(0, P_PER_W, plane, jnp.int32(-1))

    return _sc_gather


def _mm_a_body(embt_ref, w_ref, out_ref):
    out_ref[...] = jax.lax.dot_general(
        embt_ref[...], w_ref[...], (((0,), (0,)), ((), ())),
        preferred_element_type=jnp.float32)


def _mm_a_call(blk, embt_a, w0ea):
    return pl.pallas_call(
        _mm_a_body,
        grid=(B // blk,),
        in_specs=[
            pl.BlockSpec((K_HALF, blk), lambda i: (0, i)),
            pl.BlockSpec(w0ea.shape, lambda i: (0, 0)),
        ],
        out_specs=pl.BlockSpec((blk, 1024), lambda i: (i, 0)),
        out_shape=jax.ShapeDtypeStruct((B, 1024), jnp.float32),
    )(embt_a, w0ea)


def _mlp_body(xn_ref, embt_ref, ha_ref, w0n_ref, w0e_ref, b0_ref,
              w1_ref, b1_ref, w2_ref, b2_ref, w3_ref, b3_ref,
              w4_ref, b4_ref, out_ref):
    f32 = jnp.float32
    h = ha_ref[...]
    h += jnp.dot(xn_ref[...], w0n_ref[...], preferred_element_type=f32)
    h += jax.lax.dot_general(
        embt_ref[...], w0e_ref[...], (((0,), (0,)), ((), ())),
        preferred_element_type=f32)
    h = jnp.maximum(h + b0_ref[...], 0.0)
    h = jnp.maximum(
        jnp.dot(h, w1_ref[...], preferred_element_type=f32) + b1_ref[...], 0.0)
    h = jnp.maximum(
        jnp.dot(h, w2_ref[...], preferred_element_type=f32) + b2_ref[...], 0.0)
    h = jnp.maximum(
        jnp.dot(h, w3_ref[...], preferred_element_type=f32) + b3_ref[...], 0.0)
    out_ref[...] = (
        jnp.dot(h, w4_ref[...], preferred_element_type=f32) + b4_ref[...])


def _mlp_call(blk, xn, embt, ha, w0n, w0e, b0, w1, b1, w2, b2, w3, b3,
              w4p, b4p):
    n_blk = B // blk
    full = lambda a: pl.BlockSpec(a.shape, lambda i: (0,) * a.ndim)
    return pl.pallas_call(
        _mlp_body,
        grid=(n_blk,),
        in_specs=[
            pl.BlockSpec((blk, 128), lambda i: (i, 0)),
            pl.BlockSpec((K_HALF, blk), lambda i: (0, i)),
            pl.BlockSpec((blk, 1024), lambda i: (i, 0)),
            full(w0n), full(w0e), full(b0), full(w1), full(b1),
            full(w2), full(b2), full(w3), full(b3), full(w4p), full(b4p),
        ],
        out_specs=pl.BlockSpec((blk, 128), lambda i: (i, 0)),
        out_shape=jax.ShapeDtypeStruct((B, 128), jnp.float32),
    )(xn, embt, ha, w0n, w0e, b0, w1, b1, w2, b2, w3, b3, w4p, b4p)


def kernel(x_num, x_cat, tables, W0, b0, W1, b1, W2, b2, W3, b3, W4, b4):
    table2d = tables.transpose(0, 2, 1).reshape(K_EMB, VOCAB)
    xct = x_cat.astype(jnp.int32).T

    embt_a = _sc_gather_fn(0)(xct, table2d)
    embt_b = _sc_gather_fn(K_HALF)(xct, table2d)

    w0ea = W0[NUM_NUM:NUM_NUM + K_HALF]
    w0eb = W0[NUM_NUM + K_HALF:]
    ha = _mm_a_call(1024, embt_a, w0ea)

    xn = jnp.pad(x_num, ((0, 0), (0, 128 - NUM_NUM)))
    w0n = jnp.pad(W0[:NUM_NUM], ((0, 128 - NUM_NUM), (0, 0)))
    w4p = jnp.pad(W4, ((0, 0), (0, 126)))
    b4p = jnp.pad(b4, (0, 126)).reshape(1, 128)

    out = _mlp_call(
        1024, xn, embt_b, ha,
        w0n, w0eb, b0.reshape(1, -1), W1, b1.reshape(1, -1),
        W2, b2.reshape(1, -1), W3, b3.reshape(1, -1), w4p, b4p,
    )
    return out[:, :2]


# speedup vs baseline: 1.0100x; 1.0100x over previous
import functools

import jax
import jax.numpy as jnp
from jax import lax
from jax.experimental import pallas as pl
from jax.experimental.pallas import tpu as pltpu
from jax.experimental.pallas import tpu_sc as plsc

B = 16384
N_FIELDS = 26
VOCAB = 100000
EMB = 64
NUM_NUM = 13

K_EMB = N_FIELDS * EMB
NW = 32
P_PER_W = K_EMB // NW
BCH = 8192
N_BCH = B // BCH
UNROLL = 8


@functools.cache
def _sc_gather_fn():
    mesh = plsc.VectorSubcoreMesh(core_axis_name="c", subcore_axis_name="s")

    @functools.partial(
        pl.kernel,
        out_type=jax.ShapeDtypeStruct((K_EMB, B), jnp.float32),
        mesh=mesh,
        compiler_params=pltpu.CompilerParams(needs_layout_passes=False),
        scratch_types=[
            pltpu.VMEM((VOCAB,), jnp.float32),
            pltpu.VMEM((B,), jnp.int32),
            pltpu.VMEM((BCH,), jnp.float32),
        ],
    )
    def _sc_gather(xct_hbm, table_hbm, out_hbm, plane_v, idx_v, val_v):
        wid = lax.axis_index("s") * 2 + lax.axis_index("c")
        p_base = wid * P_PER_W

        def plane(p, prev_f):
            fe = p_base + p
            f = fe // EMB

            @pl.when(f != prev_f)
            def _():
                pltpu.sync_copy(xct_hbm.at[f], idx_v)

            pltpu.sync_copy(table_hbm.at[fe], plane_v)

            def bchunk(k, carry2):
                koff = pl.multiple_of(k * BCH, BCH)

                def group(g, carry3):
                    base = pl.multiple_of(g * (16 * UNROLL), 16 * UNROLL)
                    for j in range(UNROLL):
                        o = base + j * 16
                        vidx = idx_v[pl.ds(koff + o, 16)]
                        v = plsc.load_gather(plane_v, [vidx])
                        val_v[pl.ds(o, 16)] = v
                    return carry3

                lax.fori_loop(0, BCH // (16 * UNROLL), group, 0)
                pltpu.sync_copy(val_v, out_hbm.at[fe, pl.ds(koff, BCH)])
                return carry2

            lax.fori_loop(0, N_BCH, bchunk, 0)
            return f

        lax.fori_loop(0, P_PER_W, plane, jnp.int32(-1))

    return _sc_gather


def _mlp_body(xn_ref, embt_ref, w0n_ref, w0e_ref, b0_ref, w1_ref, b1_ref,
              w2_ref, b2_ref, w3_ref, b3_ref, w4_ref, b4_ref, out_ref):
    f32 = jnp.float32
    h = jnp.dot(xn_ref[...], w0n_ref[...], preferred_element_type=f32)
    h += jax.lax.dot_general(
        embt_ref[...], w0e_ref[...], (((0,), (0,)), ((), ())),
        preferred_element_type=f32)
    h = jnp.maximum(h + b0_ref[...], 0.0)
    h = jnp.maximum(
        jnp.dot(h, w1_ref[...], preferred_element_type=f32) + b1_ref[...], 0.0)
    h = jnp.maximum(
        jnp.dot(h, w2_ref[...], preferred_element_type=f32) + b2_ref[...], 0.0)
    h = jnp.maximum(
        jnp.dot(h, w3_ref[...], preferred_element_type=f32) + b3_ref[...], 0.0)
    out_ref[...] = (
        jnp.dot(h, w4_ref[...], preferred_element_type=f32) + b4_ref[...])


def _mlp_call(blk, xn, embt, w0n, w0e, b0, w1, b1, w2, b2, w3, b3,
              w4p, b4p):
    n_blk = B // blk
    full = lambda a: pl.BlockSpec(a.shape, lambda i: (0,) * a.ndim)
    return pl.pallas_call(
        _mlp_body,
        grid=(n_blk,),
        in_specs=[
            pl.BlockSpec((blk, 128), lambda i: (i, 0)),
            pl.BlockSpec((K_EMB, blk), lambda i: (0, i)),
            full(w0n), full(w0e), full(b0), full(w1), full(b1),
            full(w2), full(b2), full(w3), full(b3), full(w4p), full(b4p),
        ],
        out_specs=pl.BlockSpec((blk, 128), lambda i: (i, 0)),
        out_shape=jax.ShapeDtypeStruct((B, 128), jnp.float32),
    )(xn, embt, w0n, w0e, b0, w1, b1, w2, b2, w3, b3, w4p, b4p)


def kernel(x_num, x_cat, tables, W0, b0, W1, b1, W2, b2, W3, b3, W4, b4):
    table2d = tables.transpose(0, 2, 1).reshape(K_EMB, VOCAB)
    xct = x_cat.astype(jnp.int32).T

    embt = _sc_gather_fn()(xct, table2d)

    xn = jnp.pad(x_num, ((0, 0), (0, 128 - NUM_NUM)))
    w0n = jnp.pad(W0[:NUM_NUM], ((0, 128 - NUM_NUM), (0, 0)))
    w0e = W0[NUM_NUM:]
    w4p = jnp.pad(W4, ((0, 0), (0, 126)))
    b4p = jnp.pad(b4, (0, 126)).reshape(1, 128)

    out = _mlp_call(
        1024, xn, embt,
        w0n, w0e, b0.reshape(1, -1), W1, b1.reshape(1, -1),
        W2, b2.reshape(1, -1), W3, b3.reshape(1, -1), w4p, b4p,
    )
    return out[:, :2]
